# baseline (speedup 1.0000x reference)
import functools

import jax
import jax.numpy as jnp
from jax import lax
from jax.experimental import pallas as pl
from jax.experimental.pallas import tpu as pltpu
from jax.experimental.pallas import tpu_sc as plsc

TOP_K = 5
PROMPT_LEN = 8
PRE = (TOP_K + 1) * PROMPT_LEN + 1

NC = 2
NS = 16
NW = NC * NS
LANES = 16

B = 4
S = 2048
D = 768
RPW = S // NW


def _sc_body(x_hbm, out_hbm, idx, buf0, buf1, gs0, gs1, ss0, ss1):
    wid = lax.axis_index("s") * NC + lax.axis_index("c")
    o = jnp.minimum(56 + wid * RPW, 56 + (NW - 1) * RPW - 8)
    g0 = o - (PRE - 1)
    for j in range(RPW // LANES):
        idx[pl.ds(j * LANES, LANES)] = (
            lax.broadcasted_iota(jnp.int32, (LANES,), 0) + g0 - 1 + j * LANES)
    bufs = (buf0, buf1)
    gsems = (gs0, gs1)
    ssems = (ss0, ss1)
    gds = [None] * B
    sds = [None] * B

    def gstart(b):
        d = pltpu.make_async_copy(
            x_hbm.at[b].at[idx], bufs[b % 2], gsems[b % 2])
        d.start()
        gds[b] = d

    gstart(0)
    for b in range(B):
        if b + 1 < B:
            if b + 1 >= 2:
                sds[b - 1].wait()
            gstart(b + 1)
        gds[b].wait()
        d = pltpu.make_async_copy(
            bufs[b % 2], out_hbm.at[b, pl.ds(o, RPW), :], ssems[b % 2])
        d.start()
        sds[b] = d
    sds[B - 2].wait()
    sds[B - 1].wait()


_sc_copy = functools.partial(
    pl.kernel,
    out_type=jax.ShapeDtypeStruct((B, PRE + S, D), jnp.float32),
    mesh=plsc.VectorSubcoreMesh(core_axis_name="c", subcore_axis_name="s"),
    scratch_types=[
        pltpu.VMEM((RPW,), jnp.int32),
        pltpu.VMEM((RPW, D), jnp.float32),
        pltpu.VMEM((RPW, D), jnp.float32),
        pltpu.SemaphoreType.DMA,
        pltpu.SemaphoreType.DMA,
        pltpu.SemaphoreType.DMA,
        pltpu.SemaphoreType.DMA,
    ],
)(_sc_body)


def _tc_mean_body(x_ref, m_ref):
    m_ref[0] = jnp.sum(x_ref[0], axis=0, keepdims=True) * (1.0 / S)


def _tc_mean(x):
    return pl.pallas_call(
        _tc_mean_body,
        grid=(B,),
        in_specs=[pl.BlockSpec((1, S, D), lambda b: (b, 0, 0))],
        out_specs=pl.BlockSpec((1, 1, D), lambda b: (b, 0, 0)),
        out_shape=jax.ShapeDtypeStruct((B, 1, D), jnp.float32),
    )(x)


def _tc_prefix_body(task_ref, o_in, m_ref, x_ref, g_ref, ep_ref, ek_ref,
                    cls_ref, out_ref, head, stage, seam_sem, pf_sem):
    del o_in
    ek = ek_ref[...]
    kn = ek / jnp.maximum(
        jnp.sqrt(jnp.sum(ek * ek, axis=1, keepdims=True)), 1e-12)
    tid = task_ref[0]
    pf = []
    for b in range(B):
        hin = pltpu.make_async_copy(
            x_ref.at[b, pl.ds(0, 8), :], head.at[0], seam_sem)
        hin.start()
        tin = pltpu.make_async_copy(
            x_ref.at[b, pl.ds(S - 8, 8), :], head.at[1], seam_sem)
        tin.start()
        hin.wait()
        tin.wait()
        hv = head[0]
        tv = head[1]
        stage[0] = jnp.concatenate([cls_ref[...], hv[0:7]], axis=0)
        stage[1] = jnp.concatenate([tv[7:8], tv[0:7]], axis=0)
        d = pltpu.make_async_copy(
            stage.at[0], out_ref.at[b, pl.ds(PRE - 1, 8), :], pf_sem)
        d.start()
        pf.append(d)
        d = pltpu.make_async_copy(
            stage.at[1, pl.ds(0, 1), :],
            out_ref.at[b, pl.ds(PRE - 1 + S, 1), :], pf_sem)
        d.start()
        pf.append(d)
        q = m_ref[b]
        qn = q / jnp.maximum(jnp.sqrt(jnp.sum(q * q)), 1e-12)
        sim = jax.lax.dot_general(
            qn, kn, (((1,), (1,)), ((), ())),
            preferred_element_type=jnp.float32)
        d = pltpu.make_async_copy(
            g_ref.at[pl.ds(tid * PROMPT_LEN, PROMPT_LEN), :],
            out_ref.at[b, pl.ds(0, PROMPT_LEN), :], pf_sem)
        d.start()
        pf.append(d)
        col = lax.broadcasted_iota(jnp.int32, sim.shape, 1)
        for k in range(TOP_K):
            idx = jnp.argmax(sim[0])
            d = pltpu.make_async_copy(
                ep_ref.at[pl.ds(idx * PROMPT_LEN, PROMPT_LEN), :],
                out_ref.at[b, pl.ds((k + 1) * PROMPT_LEN, PROMPT_LEN), :],
                pf_sem)
            d.start()
            pf.append(d)
            sim = jnp.where(col == idx, -jnp.inf, sim)
    for d in pf:
        d.wait()


def kernel(x, g_prompts, e_prompts, e_keys, cls_token, task_id):
    g_flat = g_prompts.reshape(-1, D)
    ep_flat = e_prompts.reshape(-1, D)
    cls2 = cls_token.reshape(1, D)
    task = jnp.asarray(task_id, jnp.int32).reshape(1)
    out1 = _sc_copy(x)
    means = _tc_mean(x)
    return pl.pallas_call(
        _tc_prefix_body,
        in_specs=[
            pl.BlockSpec(memory_space=pltpu.MemorySpace.SMEM),
            pl.BlockSpec(memory_space=pltpu.MemorySpace.HBM),
            pl.BlockSpec(memory_space=pltpu.MemorySpace.VMEM),
            pl.BlockSpec(memory_space=pltpu.MemorySpace.HBM),
            pl.BlockSpec(memory_space=pltpu.MemorySpace.HBM),
            pl.BlockSpec(memory_space=pltpu.MemorySpace.HBM),
            pl.BlockSpec(memory_space=pltpu.MemorySpace.VMEM),
            pl.BlockSpec(memory_space=pltpu.MemorySpace.VMEM),
        ],
        out_specs=pl.BlockSpec(memory_space=pltpu.MemorySpace.HBM),
        out_shape=jax.ShapeDtypeStruct((B, PRE + S, D), jnp.float32),
        input_output_aliases={1: 0},
        scratch_shapes=[
            pltpu.VMEM((2, 8, D), jnp.float32),
            pltpu.VMEM((2, 8, D), jnp.float32),
            pltpu.SemaphoreType.DMA,
            pltpu.SemaphoreType.DMA,
        ],
    )(task, out1, means, x, g_flat, ep_flat, e_keys, cls2)


# speedup vs baseline: 1.7273x; 1.7273x over previous
import jax
import jax.numpy as jnp
from jax.experimental import pallas as pl
from jax.experimental.pallas import tpu as pltpu

TOP_K = 5
PROMPT_LEN = 8
POOL = 100


def _body(task_ref, x_ref, g_ref, ep_ref, ek_ref, cls_ref, out_ref):
    xb = x_ref[0]
    q = jnp.mean(xb, axis=0, keepdims=True)
    qn = q / jnp.maximum(jnp.sqrt(jnp.sum(q * q)), 1e-12)
    ek = ek_ref[...]
    kn = ek / jnp.maximum(
        jnp.sqrt(jnp.sum(ek * ek, axis=1, keepdims=True)), 1e-12)
    sim = jax.lax.dot_general(
        qn, kn, (((1,), (1,)), ((), ())),
        preferred_element_type=jnp.float32)

    tid = task_ref[0]
    out_ref[0, 0:PROMPT_LEN, :] = g_ref[pl.ds(tid * PROMPT_LEN, PROMPT_LEN), :]

    col = jax.lax.broadcasted_iota(jnp.int32, (1, POOL), 1)
    for k in range(TOP_K):
        idx = jnp.argmax(sim[0])
        rows = ep_ref[pl.ds(idx * PROMPT_LEN, PROMPT_LEN), :]
        base = PROMPT_LEN + k * PROMPT_LEN
        out_ref[0, base:base + PROMPT_LEN, :] = rows
        sim = jnp.where(col == idx, -jnp.inf, sim)

    ccol = (TOP_K + 1) * PROMPT_LEN
    out_ref[0, ccol:ccol + 1, :] = cls_ref[...]
    out_ref[0, ccol + 1:, :] = xb


def kernel(x, g_prompts, e_prompts, e_keys, cls_token, task_id):
    B, S, d = x.shape
    n_out = (TOP_K + 1) * PROMPT_LEN + 1 + S
    g_flat = g_prompts.reshape(-1, d)
    ep_flat = e_prompts.reshape(-1, d)
    cls2 = cls_token.reshape(1, d)
    task = jnp.asarray(task_id, jnp.int32).reshape(1)
    return pl.pallas_call(
        _body,
        grid=(B,),
        in_specs=[
            pl.BlockSpec(memory_space=pltpu.MemorySpace.SMEM),
            pl.BlockSpec((1, S, d), lambda b: (b, 0, 0)),
            pl.BlockSpec(g_flat.shape, lambda b: (0, 0)),
            pl.BlockSpec(ep_flat.shape, lambda b: (0, 0)),
            pl.BlockSpec(e_keys.shape, lambda b: (0, 0)),
            pl.BlockSpec(cls2.shape, lambda b: (0, 0)),
        ],
        out_specs=pl.BlockSpec((1, n_out, d), lambda b: (b, 0, 0)),
        out_shape=jax.ShapeDtypeStruct((B, n_out, d), x.dtype),
    )(task, x, g_flat, ep_flat, e_keys, cls2)
